# Initial kernel scaffold; baseline (speedup 1.0000x reference)
import jax
import jax.numpy as jnp
from jax.experimental import pallas as pl


def kernel(node_ids, edge_index, edge_type, emb, basis0, comp0, root0, bias0, lnw0, lnb0, basis1, comp1, root1, bias1, lnw1, lnb1, basis2, comp2, root2, bias2, lnw2, lnb2):
    raise NotImplementedError("write your pallas kernel here")


# speedup vs baseline: 1.1069x; 1.1069x over previous
import functools

import jax
import jax.numpy as jnp
from jax import lax
from jax.experimental import pallas as pl
from jax.experimental.pallas import tpu as pltpu

N = 10000
E = 320000
R = 16
NB = 16
D = 128
BN = 1000


def _y_body(comp_ref, x_ref, basis_ref, y_ref):
    r = pl.program_id(1)
    rows = lax.broadcasted_iota(jnp.int32, (R, 1), 0)
    comp_r = jnp.sum(jnp.where(rows == r, comp_ref[...], 0.0), axis=0,
                     keepdims=True)
    w_r = jnp.dot(comp_r, basis_ref[...].reshape(NB, D * D),
                  preferred_element_type=jnp.float32).reshape(D, D)
    y_ref[0] = jnp.dot(x_ref[...], w_r, preferred_element_type=jnp.float32)


def _y_kernel(x, basis, comp):
    return pl.pallas_call(
        _y_body,
        grid=(N // BN, R),
        in_specs=[
            pl.BlockSpec((R, NB), lambda n, r: (0, 0)),
            pl.BlockSpec((BN, D), lambda n, r: (n, 0)),
            pl.BlockSpec((NB, D, D), lambda n, r: (0, 0, 0)),
        ],
        out_specs=pl.BlockSpec((1, BN, D), lambda n, r: (r, n, 0)),
        out_shape=jax.ShapeDtypeStruct((R, N, D), jnp.float32),
    )(comp, x, basis)


def _combine_body(x_ref, agg_ref, root_ref, bias_ref, lnw_ref, lnb_ref, out_ref):
    agg = agg_ref[0] + agg_ref[1]
    t = agg + jnp.dot(x_ref[...], root_ref[...],
                      preferred_element_type=jnp.float32) + bias_ref[...]
    mu = jnp.mean(t, axis=-1, keepdims=True)
    var = jnp.mean((t - mu) ** 2, axis=-1, keepdims=True)
    t = (t - mu) * lax.rsqrt(var + 1e-5) * lnw_ref[...] + lnb_ref[...]
    out_ref[...] = x_ref[...] + jnp.maximum(t, 0.0)


def _combine_kernel(x, agg2, root, bias, lnw, lnb):
    return pl.pallas_call(
        _combine_body,
        grid=(N // BN,),
        in_specs=[
            pl.BlockSpec((BN, D), lambda n: (n, 0)),
            pl.BlockSpec((2, BN, D), lambda n: (0, n, 0)),
            pl.BlockSpec((D, D), lambda n: (0, 0)),
            pl.BlockSpec((1, D), lambda n: (0, 0)),
            pl.BlockSpec((1, D), lambda n: (0, 0)),
            pl.BlockSpec((1, D), lambda n: (0, 0)),
        ],
        out_specs=pl.BlockSpec((BN, D), lambda n: (n, 0)),
        out_shape=jax.ShapeDtypeStruct((N, D), jnp.float32),
    )(x, agg2, root, bias.reshape(1, D), lnw.reshape(1, D), lnb.reshape(1, D))


def _mean_body(x_ref, out_ref):
    @pl.when(pl.program_id(0) == 0)
    def _init():
        out_ref[...] = jnp.zeros_like(out_ref)
    out_ref[...] += jnp.sum(x_ref[...], axis=0, keepdims=True) * (1.0 / N)


def _mean_kernel(x):
    return pl.pallas_call(
        _mean_body,
        grid=(N // BN,),
        in_specs=[pl.BlockSpec((BN, D), lambda n: (n, 0))],
        out_specs=pl.BlockSpec((1, D), lambda n: (0, 0)),
        out_shape=jax.ShapeDtypeStruct((1, D), jnp.float32),
    )(x)


def kernel(node_ids, edge_index, edge_type, emb,
           basis0, comp0, root0, bias0, lnw0, lnb0,
           basis1, comp1, root1, bias1, lnw1, lnb1,
           basis2, comp2, root2, bias2, lnw2, lnb2):
    src = edge_index[0]
    dst = edge_index[1]
    etype = edge_type
    seg = dst * R + etype
    gidx = etype * N + src

    cnt = jnp.zeros((N * R,), jnp.float32).at[seg].add(1.0)
    w_edge = (1.0 / jnp.maximum(cnt, 1.0))[seg]

    x = jnp.take(emb, node_ids, axis=0)

    layers = [(basis0, comp0, root0, bias0, lnw0, lnb0),
              (basis1, comp1, root1, bias1, lnw1, lnb1),
              (basis2, comp2, root2, bias2, lnw2, lnb2)]
    zero_half = jnp.zeros((1, N, D), jnp.float32)
    for (ba, co, ro, bi, lw, lb) in layers:
        y = _y_kernel(x, ba, co).reshape(R * N, D)
        msg = jnp.take(y, gidx, axis=0) * w_edge[:, None]
        agg = jnp.zeros((N, D), jnp.float32).at[dst].add(msg)
        agg2 = jnp.concatenate([agg[None], zero_half], axis=0)
        x = _combine_kernel(x, agg2, ro, bi, lw, lb)

    return (x, _mean_kernel(x))


# speedup vs baseline: 2.4624x; 2.2246x over previous
import functools

import jax
import jax.numpy as jnp
from jax import lax
from jax.experimental import pallas as pl
from jax.experimental.pallas import tpu as pltpu
from jax.experimental.pallas import tpu_sc as plsc

N = 10000
E = 320000
R = 16
NB = 16
D = 128
BN = 1000

NTILES = 32
CH = 128
EPT = 10240
EPAD = NTILES * EPT
EPC = EPAD // 16
NRP = 163840
NACC = 10240
NPAD = NTILES * 320

_MESH = plsc.VectorSubcoreMesh(core_axis_name="c", subcore_axis_name="s")


def _zero_vmem_1d(buf, nvec):
    z = jnp.zeros((16,), jnp.float32)

    def body(i, _):
        buf[pl.ds(i * 16, 16)] = z
        return 0

    lax.fori_loop(0, nvec, body, 0)


def _zero_vmem_2d(buf, nrows):
    z = jnp.zeros((16,), jnp.float32)

    def body(i, _):
        for dd in range(D // 16):
            buf[i, pl.ds(dd * 16, 16)] = z
        return 0

    lax.fori_loop(0, nrows, body, 0)


@functools.partial(
    pl.kernel,
    out_type=jax.ShapeDtypeStruct((NPAD, D), jnp.float32),
    mesh=_MESH,
    scratch_types=[
        pltpu.VMEM((80,), jnp.int32),
        pltpu.VMEM((80, D), jnp.float32),
        pltpu.SemaphoreType.DMA,
    ],
)
def _sc_x0(emb_hbm, ids_hbm, out_hbm, idxb, rows, sem):
    c = lax.axis_index("c")
    s = lax.axis_index("s")
    wid = c * 16 + s

    def chunk(j, _):
        base = wid * 320 + j * 80
        pltpu.sync_copy(ids_hbm.at[pl.ds(base, 80)], idxb)
        pltpu.async_copy(emb_hbm.at[idxb], rows, sem).wait()
        pltpu.sync_copy(rows, out_hbm.at[pl.ds(base, 80)])
        return 0

    lax.fori_loop(0, 4, chunk, 0)


@functools.partial(
    pl.kernel,
    out_type=jax.ShapeDtypeStruct((EPAD,), jnp.float32),
    mesh=_MESH,
    scratch_types=[
        pltpu.VMEM_SHARED((NRP,), jnp.float32),
        pltpu.VMEM((CH,), jnp.int32),
        pltpu.VMEM((CH,), jnp.float32),
        pltpu.VMEM((2048,), jnp.float32),
        pltpu.SemaphoreType.DMA,
    ],
)
def _sc_weights(seg_hbm, w_hbm, cnt, segb, wb, stage, sem):
    c = lax.axis_index("c")
    s = lax.axis_index("s")
    wid = c * 16 + s

    def ones_body(i, _):
        wb[pl.ds(i * 16, 16)] = jnp.full((16,), 1.0, jnp.float32)
        return 0

    lax.fori_loop(0, CH // 16, ones_body, 0)

    _zero_vmem_1d(stage, 128)

    def zchunk(j, _):
        pltpu.sync_copy(stage, cnt.at[pl.ds(s * (NRP // 16) + j * 2048, 2048)])
        return 0

    lax.fori_loop(0, NRP // 16 // 2048, zchunk, 0)
    plsc.subcore_barrier()

    def cchunk(k, _):
        pltpu.sync_copy(seg_hbm.at[pl.ds(s * EPC + k * CH, CH)], segb)
        pltpu.sync_copy(wb, cnt.at[segb], add=True)
        return 0

    lax.fori_loop(0, EPC // CH, cchunk, 0)
    plsc.subcore_barrier()

    def ichunk(j, _):
        base = s * (NRP // 16) + j * 2048
        pltpu.sync_copy(cnt.at[pl.ds(base, 2048)], stage)

        def inv(i, _):
            v = stage[pl.ds(i * 16, 16)]
            stage[pl.ds(i * 16, 16)] = 1.0 / jnp.maximum(v, 1.0)
            return 0

        lax.fori_loop(0, 128, inv, 0)
        pltpu.sync_copy(stage, cnt.at[pl.ds(base, 2048)])
        return 0

    lax.fori_loop(0, NRP // 16 // 2048, ichunk, 0)
    plsc.subcore_barrier()

    def gchunk(k, _):
        base = wid * EPT + k * CH
        pltpu.sync_copy(seg_hbm.at[pl.ds(base, CH)], segb)
        pltpu.async_copy(cnt.at[segb], wb, sem).wait()
        pltpu.sync_copy(wb, w_hbm.at[pl.ds(base, CH)])
        return 0

    lax.fori_loop(0, EPT // CH, gchunk, 0)


@functools.partial(
    pl.kernel,
    out_type=jax.ShapeDtypeStruct((2, NACC, D), jnp.float32),
    mesh=_MESH,
    scratch_types=[
        pltpu.VMEM_SHARED((NACC, D), jnp.float32),
        pltpu.VMEM((CH, D), jnp.float32),
        pltpu.VMEM((CH,), jnp.int32),
        pltpu.VMEM((CH,), jnp.int32),
        pltpu.VMEM((CH,), jnp.float32),
        pltpu.VMEM((128, D), jnp.float32),
        pltpu.SemaphoreType.DMA,
    ],
)
def _sc_edge(y_hbm, gidx_hbm, dst_hbm, w_hbm, out_hbm,
             acc, rows, idxb, dstb, wb, zb, sem):
    c = lax.axis_index("c")
    s = lax.axis_index("s")
    wid = c * 16 + s

    _zero_vmem_2d(zb, 128)

    def zchunk(j, _):
        pltpu.sync_copy(zb, acc.at[pl.ds(s * 640 + j * 128, 128)])
        return 0

    lax.fori_loop(0, 5, zchunk, 0)
    plsc.subcore_barrier()

    def chunk(k, _):
        base = wid * EPT + k * CH
        pltpu.sync_copy(gidx_hbm.at[pl.ds(base, CH)], idxb)
        pltpu.sync_copy(dst_hbm.at[pl.ds(base, CH)], dstb)
        pltpu.sync_copy(w_hbm.at[pl.ds(base, CH)], wb)
        pltpu.async_copy(y_hbm.at[idxb], rows, sem).wait()

        def scale(g, _):
            wv = wb[pl.ds(g * 16, 16)]
            for j in range(16):
                e = g * 16 + j
                w = jnp.broadcast_to(wv[j], (16,))
                for dd in range(D // 16):
                    sl = pl.ds(dd * 16, 16)
                    rows[e, sl] = rows[e, sl] * w
            return 0

        lax.fori_loop(0, CH // 16, scale, 0)
        pltpu.sync_copy(rows, acc.at[dstb], add=True)
        return 0

    lax.fori_loop(0, EPT // CH, chunk, 0)
    plsc.subcore_barrier()

    pltpu.sync_copy(acc.at[pl.ds(s * 640, 640)],
                    out_hbm.at[c].at[pl.ds(s * 640, 640)])


def _y_body(comp_ref, x_ref, basis_ref, y_ref):
    r = pl.program_id(1)
    rows = lax.broadcasted_iota(jnp.int32, (R, 1), 0)
    comp_r = jnp.sum(jnp.where(rows == r, comp_ref[...], 0.0), axis=0,
                     keepdims=True)
    w_r = jnp.dot(comp_r, basis_ref[...].reshape(NB, D * D),
                  preferred_element_type=jnp.float32).reshape(D, D)
    y_ref[0] = jnp.dot(x_ref[...], w_r, preferred_element_type=jnp.float32)


def _y_kernel(x, basis, comp):
    return pl.pallas_call(
        _y_body,
        grid=(N // BN, R),
        in_specs=[
            pl.BlockSpec((R, NB), lambda n, r: (0, 0)),
            pl.BlockSpec((BN, D), lambda n, r: (n, 0)),
            pl.BlockSpec((NB, D, D), lambda n, r: (0, 0, 0)),
        ],
        out_specs=pl.BlockSpec((1, BN, D), lambda n, r: (r, n, 0)),
        out_shape=jax.ShapeDtypeStruct((R, N, D), jnp.float32),
    )(comp, x, basis)


def _combine_body(x_ref, agg_ref, root_ref, bias_ref, lnw_ref, lnb_ref, out_ref):
    agg = agg_ref[0] + agg_ref[1]
    t = agg + jnp.dot(x_ref[...], root_ref[...],
                      preferred_element_type=jnp.float32) + bias_ref[...]
    mu = jnp.mean(t, axis=-1, keepdims=True)
    var = jnp.mean((t - mu) ** 2, axis=-1, keepdims=True)
    t = (t - mu) * lax.rsqrt(var + 1e-5) * lnw_ref[...] + lnb_ref[...]
    out_ref[...] = x_ref[...] + jnp.maximum(t, 0.0)


def _combine_kernel(x, agg2, root, bias, lnw, lnb):
    return pl.pallas_call(
        _combine_body,
        grid=(N // BN,),
        in_specs=[
            pl.BlockSpec((BN, D), lambda n: (n, 0)),
            pl.BlockSpec((2, BN, D), lambda n: (0, n, 0)),
            pl.BlockSpec((D, D), lambda n: (0, 0)),
            pl.BlockSpec((1, D), lambda n: (0, 0)),
            pl.BlockSpec((1, D), lambda n: (0, 0)),
            pl.BlockSpec((1, D), lambda n: (0, 0)),
        ],
        out_specs=pl.BlockSpec((BN, D), lambda n: (n, 0)),
        out_shape=jax.ShapeDtypeStruct((N, D), jnp.float32),
    )(x, agg2, root, bias.reshape(1, D), lnw.reshape(1, D), lnb.reshape(1, D))


def _mean_body(x_ref, out_ref):
    @pl.when(pl.program_id(0) == 0)
    def _init():
        out_ref[...] = jnp.zeros_like(out_ref)
    out_ref[...] += jnp.sum(x_ref[...], axis=0, keepdims=True) * (1.0 / N)


def _mean_kernel(x):
    return pl.pallas_call(
        _mean_body,
        grid=(N // BN,),
        in_specs=[pl.BlockSpec((BN, D), lambda n: (n, 0))],
        out_specs=pl.BlockSpec((1, D), lambda n: (0, 0)),
        out_shape=jax.ShapeDtypeStruct((1, D), jnp.float32),
    )(x)


def kernel(node_ids, edge_index, edge_type, emb,
           basis0, comp0, root0, bias0, lnw0, lnb0,
           basis1, comp1, root1, bias1, lnw1, lnb1,
           basis2, comp2, root2, bias2, lnw2, lnb2):
    src = edge_index[0]
    dst = edge_index[1]
    etype = edge_type

    pad = EPAD - E
    seg = jnp.concatenate([dst * R + etype,
                           jnp.full((pad,), N * R, jnp.int32)])
    gidx = jnp.concatenate([etype * N + src, jnp.zeros((pad,), jnp.int32)])
    dstp = jnp.concatenate([dst, jnp.full((pad,), N, jnp.int32)])
    ids = jnp.concatenate([node_ids,
                           jnp.zeros((NPAD - N,), node_ids.dtype)])

    w_edge = _sc_weights(seg)
    x = _sc_x0(emb, ids)[:N]

    layers = [(basis0, comp0, root0, bias0, lnw0, lnb0),
              (basis1, comp1, root1, bias1, lnw1, lnb1),
              (basis2, comp2, root2, bias2, lnw2, lnb2)]
    for (ba, co, ro, bi, lw, lb) in layers:
        y = _y_kernel(x, ba, co).reshape(R * N, D)
        agg2 = _sc_edge(y, gidx, dstp, w_edge)
        x = _combine_kernel(x, agg2, ro, bi, lw, lb)

    return (x, _mean_kernel(x))


# speedup vs baseline: 2.9278x; 1.1890x over previous
import functools

import jax
import jax.numpy as jnp
from jax import lax
from jax.experimental import pallas as pl
from jax.experimental.pallas import tpu as pltpu
from jax.experimental.pallas import tpu_sc as plsc

N = 10000
E = 320000
R = 16
NB = 16
D = 128
BN = 1000

NTILES = 32
CH = 128
EPT = 10240
EPAD = NTILES * EPT
EPC = EPAD // 16
NRP = 163840
NACC = 10240
NPAD = NTILES * 320

_MESH = plsc.VectorSubcoreMesh(core_axis_name="c", subcore_axis_name="s")


def _zero_vmem_1d(buf, nvec):
    z = jnp.zeros((16,), jnp.float32)

    def body(i, _):
        buf[pl.ds(i * 16, 16)] = z
        return 0

    lax.fori_loop(0, nvec, body, 0)


def _zero_vmem_2d(buf, nrows):
    z = jnp.zeros((16,), jnp.float32)

    def body(i, _):
        for dd in range(D // 16):
            buf[i, pl.ds(dd * 16, 16)] = z
        return 0

    lax.fori_loop(0, nrows, body, 0)


@functools.partial(
    pl.kernel,
    out_type=jax.ShapeDtypeStruct((NPAD, D), jnp.float32),
    mesh=_MESH,
    scratch_types=[
        pltpu.VMEM((80,), jnp.int32),
        pltpu.VMEM((80, D), jnp.float32),
        pltpu.SemaphoreType.DMA,
    ],
)
def _sc_x0(emb_hbm, ids_hbm, out_hbm, idxb, rows, sem):
    c = lax.axis_index("c")
    s = lax.axis_index("s")
    wid = c * 16 + s

    def chunk(j, _):
        base = wid * 320 + j * 80
        pltpu.sync_copy(ids_hbm.at[pl.ds(base, 80)], idxb)
        pltpu.async_copy(emb_hbm.at[idxb], rows, sem).wait()
        pltpu.sync_copy(rows, out_hbm.at[pl.ds(base, 80)])
        return 0

    lax.fori_loop(0, 4, chunk, 0)


@functools.partial(
    pl.kernel,
    out_type=jax.ShapeDtypeStruct((EPAD,), jnp.float32),
    mesh=_MESH,
    scratch_types=[
        pltpu.VMEM_SHARED((NRP,), jnp.float32),
        pltpu.VMEM((CH,), jnp.int32),
        pltpu.VMEM((CH,), jnp.float32),
        pltpu.VMEM((2048,), jnp.float32),
        pltpu.SemaphoreType.DMA,
    ],
)
def _sc_weights(seg_hbm, w_hbm, cnt, segb, wb, stage, sem):
    c = lax.axis_index("c")
    s = lax.axis_index("s")
    wid = c * 16 + s

    def ones_body(i, _):
        wb[pl.ds(i * 16, 16)] = jnp.full((16,), 1.0, jnp.float32)
        return 0

    lax.fori_loop(0, CH // 16, ones_body, 0)

    _zero_vmem_1d(stage, 128)

    def zchunk(j, _):
        pltpu.sync_copy(stage, cnt.at[pl.ds(s * (NRP // 16) + j * 2048, 2048)])
        return 0

    lax.fori_loop(0, NRP // 16 // 2048, zchunk, 0)
    plsc.subcore_barrier()

    def cchunk(k, _):
        pltpu.sync_copy(seg_hbm.at[pl.ds(s * EPC + k * CH, CH)], segb)
        pltpu.sync_copy(wb, cnt.at[segb], add=True)
        return 0

    lax.fori_loop(0, EPC // CH, cchunk, 0)
    plsc.subcore_barrier()

    def ichunk(j, _):
        base = s * (NRP // 16) + j * 2048
        pltpu.sync_copy(cnt.at[pl.ds(base, 2048)], stage)

        def inv(i, _):
            v = stage[pl.ds(i * 16, 16)]
            stage[pl.ds(i * 16, 16)] = 1.0 / jnp.maximum(v, 1.0)
            return 0

        lax.fori_loop(0, 128, inv, 0)
        pltpu.sync_copy(stage, cnt.at[pl.ds(base, 2048)])
        return 0

    lax.fori_loop(0, NRP // 16 // 2048, ichunk, 0)
    plsc.subcore_barrier()

    def gchunk(k, _):
        base = wid * EPT + k * CH
        pltpu.sync_copy(seg_hbm.at[pl.ds(base, CH)], segb)
        pltpu.async_copy(cnt.at[segb], wb, sem).wait()
        pltpu.sync_copy(wb, w_hbm.at[pl.ds(base, CH)])
        return 0

    lax.fori_loop(0, EPT // CH, gchunk, 0)


NCHT = EPT // CH
RBYTES = CH * D * 4


@functools.partial(
    pl.kernel,
    out_type=jax.ShapeDtypeStruct((2, NACC, D), jnp.float32),
    mesh=_MESH,
    scratch_types=[
        pltpu.VMEM_SHARED((NACC, D), jnp.float32),
        pltpu.VMEM((2, CH, D), jnp.float32),
        pltpu.VMEM((4, 2, CH), jnp.int32),
        pltpu.VMEM((4, CH), jnp.float32),
        pltpu.SemaphoreType.DMA((4,)),
        pltpu.SemaphoreType.DMA((2,)),
        pltpu.SemaphoreType.DMA((2,)),
    ],
)
def _sc_edge(y_hbm, edata_hbm, w_hbm, out_hbm, acc, rows, ebuf, wbuf,
             csem, gsem, ssem):
    c = lax.axis_index("c")
    s = lax.axis_index("s")
    wid = c * 16 + s
    cbase = wid * NCHT

    def fire_c(k):
        q = k % 4
        pltpu.async_copy(edata_hbm.at[cbase + k], ebuf.at[q], csem.at[q])
        pltpu.async_copy(w_hbm.at[pl.ds((cbase + k) * CH, CH)], wbuf.at[q],
                         csem.at[q])

    def fire_g(k):
        pltpu.async_copy(y_hbm.at[ebuf.at[k % 4, 0]], rows.at[k % 2],
                         gsem.at[k % 2])

    def wait_c(q):
        pltpu.make_async_copy(edata_hbm.at[cbase], ebuf.at[q],
                              csem.at[q]).wait()
        pltpu.make_async_copy(w_hbm.at[pl.ds(0, CH)], wbuf.at[q],
                              csem.at[q]).wait()

    def wait_g(p):
        pltpu.make_async_copy(y_hbm.at[pl.ds(0, CH)], rows.at[p],
                              gsem.at[p]).wait()

    def wait_s(p):
        pltpu.make_async_copy(y_hbm.at[pl.ds(0, CH)], rows.at[p],
                              ssem.at[p]).wait()

    fire_c(0)
    fire_c(1)
    _zero_vmem_2d(rows.at[0], CH)

    def zchunk(j, _):
        pltpu.sync_copy(rows.at[0], acc.at[pl.ds(s * 640 + j * 128, 128)])
        return 0

    lax.fori_loop(0, 5, zchunk, 0)
    plsc.subcore_barrier()

    def body(k, fire):
        p = k % 2
        p1 = (k + 1) % 2
        q = k % 4

        @pl.when(k >= 1)
        def _ws():
            wait_s(p1)
        if fire:
            wait_c((k + 1) % 4)
            fire_g(k + 1)
            fire_c(k + 2)
        wait_g(p)

        def scale(g, _):
            wv = wbuf[q, pl.ds(g * 16, 16)]
            for j in range(16):
                e = g * 16 + j
                w = jnp.broadcast_to(wv[j], (16,))
                for dd in range(D // 16):
                    sl = pl.ds(dd * 16, 16)
                    rows[p, e, sl] = rows[p, e, sl] * w
            return 0

        lax.fori_loop(0, CH // 16, scale, 0)
        pltpu.async_copy(rows.at[p], acc.at[ebuf.at[q, 1]], ssem.at[p],
                         add=True)
        return 0

    wait_c(0)
    fire_g(0)
    lax.fori_loop(0, NCHT - 2, lambda k, _: body(k, True), 0)
    body(jnp.int32(NCHT - 2), False)
    wait_c((NCHT - 1) % 4)
    fire_g(NCHT - 1)
    body(jnp.int32(NCHT - 1), False)
    wait_s((NCHT - 1) % 2)
    plsc.subcore_barrier()

    pltpu.sync_copy(acc.at[pl.ds(s * 640, 640)],
                    out_hbm.at[c].at[pl.ds(s * 640, 640)])


def _y_body(comp_ref, x_ref, basis_ref, y_ref):
    r = pl.program_id(1)
    rows = lax.broadcasted_iota(jnp.int32, (R, 1), 0)
    comp_r = jnp.sum(jnp.where(rows == r, comp_ref[...], 0.0), axis=0,
                     keepdims=True)
    w_r = jnp.dot(comp_r, basis_ref[...].reshape(NB, D * D),
                  preferred_element_type=jnp.float32).reshape(D, D)
    y_ref[0] = jnp.dot(x_ref[...], w_r, preferred_element_type=jnp.float32)


def _y_kernel(x, basis, comp):
    return pl.pallas_call(
        _y_body,
        grid=(N // BN, R),
        in_specs=[
            pl.BlockSpec((R, NB), lambda n, r: (0, 0)),
            pl.BlockSpec((BN, D), lambda n, r: (n, 0)),
            pl.BlockSpec((NB, D, D), lambda n, r: (0, 0, 0)),
        ],
        out_specs=pl.BlockSpec((1, BN, D), lambda n, r: (r, n, 0)),
        out_shape=jax.ShapeDtypeStruct((R, N, D), jnp.float32),
    )(comp, x, basis)


def _combine_body(x_ref, agg_ref, root_ref, bias_ref, lnw_ref, lnb_ref, out_ref):
    agg = agg_ref[0] + agg_ref[1]
    t = agg + jnp.dot(x_ref[...], root_ref[...],
                      preferred_element_type=jnp.float32) + bias_ref[...]
    mu = jnp.mean(t, axis=-1, keepdims=True)
    var = jnp.mean((t - mu) ** 2, axis=-1, keepdims=True)
    t = (t - mu) * lax.rsqrt(var + 1e-5) * lnw_ref[...] + lnb_ref[...]
    out_ref[...] = x_ref[...] + jnp.maximum(t, 0.0)


def _combine_kernel(x, agg2, root, bias, lnw, lnb):
    return pl.pallas_call(
        _combine_body,
        grid=(N // BN,),
        in_specs=[
            pl.BlockSpec((BN, D), lambda n: (n, 0)),
            pl.BlockSpec((2, BN, D), lambda n: (0, n, 0)),
            pl.BlockSpec((D, D), lambda n: (0, 0)),
            pl.BlockSpec((1, D), lambda n: (0, 0)),
            pl.BlockSpec((1, D), lambda n: (0, 0)),
            pl.BlockSpec((1, D), lambda n: (0, 0)),
        ],
        out_specs=pl.BlockSpec((BN, D), lambda n: (n, 0)),
        out_shape=jax.ShapeDtypeStruct((N, D), jnp.float32),
    )(x, agg2, root, bias.reshape(1, D), lnw.reshape(1, D), lnb.reshape(1, D))


def _mean_body(x_ref, out_ref):
    @pl.when(pl.program_id(0) == 0)
    def _init():
        out_ref[...] = jnp.zeros_like(out_ref)
    out_ref[...] += jnp.sum(x_ref[...], axis=0, keepdims=True) * (1.0 / N)


def _mean_kernel(x):
    return pl.pallas_call(
        _mean_body,
        grid=(N // BN,),
        in_specs=[pl.BlockSpec((BN, D), lambda n: (n, 0))],
        out_specs=pl.BlockSpec((1, D), lambda n: (0, 0)),
        out_shape=jax.ShapeDtypeStruct((1, D), jnp.float32),
    )(x)


def kernel(node_ids, edge_index, edge_type, emb,
           basis0, comp0, root0, bias0, lnw0, lnb0,
           basis1, comp1, root1, bias1, lnw1, lnb1,
           basis2, comp2, root2, bias2, lnw2, lnb2):
    src = edge_index[0]
    dst = edge_index[1]
    etype = edge_type

    pad = EPAD - E
    seg = jnp.concatenate([dst * R + etype,
                           jnp.full((pad,), N * R, jnp.int32)])
    gidx = jnp.concatenate([etype * N + src, jnp.zeros((pad,), jnp.int32)])
    dstp = jnp.concatenate([dst, jnp.full((pad,), N, jnp.int32)])
    ids = jnp.concatenate([node_ids,
                           jnp.zeros((NPAD - N,), node_ids.dtype)])

    w_edge = _sc_weights(seg)
    x = _sc_x0(emb, ids)[:N]

    edata = jnp.stack([gidx.reshape(-1, CH), dstp.reshape(-1, CH)], axis=1)

    layers = [(basis0, comp0, root0, bias0, lnw0, lnb0),
              (basis1, comp1, root1, bias1, lnw1, lnb1),
              (basis2, comp2, root2, bias2, lnw2, lnb2)]
    for (ba, co, ro, bi, lw, lb) in layers:
        y = _y_kernel(x, ba, co).reshape(R * N, D)
        agg2 = _sc_edge(y, edata, w_edge)
        x = _combine_kernel(x, agg2, ro, bi, lw, lb)

    return (x, _mean_kernel(x))


# speedup vs baseline: 3.0808x; 1.0523x over previous
import functools

import jax
import jax.numpy as jnp
from jax import lax
from jax.experimental import pallas as pl
from jax.experimental.pallas import tpu as pltpu
from jax.experimental.pallas import tpu_sc as plsc

N = 10000
E = 320000
R = 16
NB = 16
D = 128
BN = 1000

NTILES = 32
CH = 80
EPT = 10240
EPAD = NTILES * EPT
EPC = EPAD // 16
NRP = 163840
NACC = 10240
NPAD = NTILES * 320

_MESH = plsc.VectorSubcoreMesh(core_axis_name="c", subcore_axis_name="s")


def _zero_vmem_1d(buf, nvec):
    z = jnp.zeros((16,), jnp.float32)

    def body(i, _):
        buf[pl.ds(i * 16, 16)] = z
        return 0

    lax.fori_loop(0, nvec, body, 0)


def _zero_vmem_2d(buf, nrows):
    z = jnp.zeros((16,), jnp.float32)

    def body(i, _):
        for dd in range(D // 16):
            buf[i, pl.ds(dd * 16, 16)] = z
        return 0

    lax.fori_loop(0, nrows, body, 0)


@functools.partial(
    pl.kernel,
    out_type=jax.ShapeDtypeStruct((NPAD, D), jnp.float32),
    mesh=_MESH,
    scratch_types=[
        pltpu.VMEM((80,), jnp.int32),
        pltpu.VMEM((80, D), jnp.float32),
        pltpu.SemaphoreType.DMA,
    ],
)
def _sc_x0(emb_hbm, ids_hbm, out_hbm, idxb, rows, sem):
    c = lax.axis_index("c")
    s = lax.axis_index("s")
    wid = c * 16 + s

    def chunk(j, _):
        base = wid * 320 + j * 80
        pltpu.sync_copy(ids_hbm.at[pl.ds(base, 80)], idxb)
        pltpu.async_copy(emb_hbm.at[idxb], rows, sem).wait()
        pltpu.sync_copy(rows, out_hbm.at[pl.ds(base, 80)])
        return 0

    lax.fori_loop(0, 4, chunk, 0)


@functools.partial(
    pl.kernel,
    out_type=jax.ShapeDtypeStruct((EPAD,), jnp.float32),
    mesh=_MESH,
    scratch_types=[
        pltpu.VMEM_SHARED((NRP,), jnp.float32),
        pltpu.VMEM((CH,), jnp.int32),
        pltpu.VMEM((CH,), jnp.float32),
        pltpu.VMEM((2048,), jnp.float32),
        pltpu.SemaphoreType.DMA,
    ],
)
def _sc_weights(seg_hbm, w_hbm, cnt, segb, wb, stage, sem):
    c = lax.axis_index("c")
    s = lax.axis_index("s")
    wid = c * 16 + s

    def ones_body(i, _):
        wb[pl.ds(i * 16, 16)] = jnp.full((16,), 1.0, jnp.float32)
        return 0

    lax.fori_loop(0, CH // 16, ones_body, 0)

    _zero_vmem_1d(stage, 128)

    def zchunk(j, _):
        pltpu.sync_copy(stage, cnt.at[pl.ds(s * (NRP // 16) + j * 2048, 2048)])
        return 0

    lax.fori_loop(0, NRP // 16 // 2048, zchunk, 0)
    plsc.subcore_barrier()

    def cchunk(k, _):
        pltpu.sync_copy(seg_hbm.at[pl.ds(s * EPC + k * CH, CH)], segb)
        pltpu.sync_copy(wb, cnt.at[segb], add=True)
        return 0

    lax.fori_loop(0, EPC // CH, cchunk, 0)
    plsc.subcore_barrier()

    def ichunk(j, _):
        base = s * (NRP // 16) + j * 2048
        pltpu.sync_copy(cnt.at[pl.ds(base, 2048)], stage)

        def inv(i, _):
            v = stage[pl.ds(i * 16, 16)]
            stage[pl.ds(i * 16, 16)] = 1.0 / jnp.maximum(v, 1.0)
            return 0

        lax.fori_loop(0, 128, inv, 0)
        pltpu.sync_copy(stage, cnt.at[pl.ds(base, 2048)])
        return 0

    lax.fori_loop(0, NRP // 16 // 2048, ichunk, 0)
    plsc.subcore_barrier()

    def gchunk(k, _):
        base = wid * EPT + k * CH
        pltpu.sync_copy(seg_hbm.at[pl.ds(base, CH)], segb)
        pltpu.async_copy(cnt.at[segb], wb, sem).wait()
        pltpu.sync_copy(wb, w_hbm.at[pl.ds(base, CH)])
        return 0

    lax.fori_loop(0, EPT // CH, gchunk, 0)


NCHT = EPT // CH
RBYTES = CH * D * 4


@functools.partial(
    pl.kernel,
    out_type=jax.ShapeDtypeStruct((2, NACC, D), jnp.float32),
    mesh=_MESH,
    scratch_types=[
        pltpu.VMEM_SHARED((NACC, D), jnp.float32),
        pltpu.VMEM((4, CH, D), jnp.float32),
        pltpu.VMEM((8, 2, CH), jnp.int32),
        pltpu.VMEM((8, CH), jnp.float32),
        pltpu.SemaphoreType.DMA((8,)),
        pltpu.SemaphoreType.DMA((4,)),
        pltpu.SemaphoreType.DMA((4,)),
    ],
)
def _sc_edge(y_hbm, edata_hbm, w_hbm, out_hbm, acc, rows, ebuf, wbuf,
             csem, gsem, ssem):
    c = lax.axis_index("c")
    s = lax.axis_index("s")
    wid = c * 16 + s
    cbase = wid * NCHT

    def fire_c(k):
        q = k % 8
        pltpu.async_copy(edata_hbm.at[cbase + k], ebuf.at[q], csem.at[q])
        pltpu.async_copy(w_hbm.at[pl.ds((cbase + k) * CH, CH)], wbuf.at[q],
                         csem.at[q])

    def fire_g(k):
        pltpu.async_copy(y_hbm.at[ebuf.at[k % 8, 0]], rows.at[k % 4],
                         gsem.at[k % 4])

    def wait_c(q):
        pltpu.make_async_copy(edata_hbm.at[cbase], ebuf.at[q],
                              csem.at[q]).wait()
        pltpu.make_async_copy(w_hbm.at[pl.ds(0, CH)], wbuf.at[q],
                              csem.at[q]).wait()

    def wait_g(p):
        pltpu.make_async_copy(y_hbm.at[pl.ds(0, CH)], rows.at[p],
                              gsem.at[p]).wait()

    def wait_s(p):
        pltpu.make_async_copy(y_hbm.at[pl.ds(0, CH)], rows.at[p],
                              ssem.at[p]).wait()

    for j in range(5):
        fire_c(j)
    _zero_vmem_2d(rows.at[0], CH)

    def zchunk(j, _):
        pltpu.sync_copy(rows.at[0], acc.at[pl.ds(s * 640 + j * CH, CH)])
        return 0

    lax.fori_loop(0, 640 // CH, zchunk, 0)
    plsc.subcore_barrier()

    def body(k, _):
        p = k % 4
        q = k % 8
        p2 = (k + 2) % 4

        @pl.when(k >= 2)
        def _ws():
            wait_s(p2)

        @pl.when(k < NCHT - 2)
        def _fg():
            wait_c((k + 2) % 8)
            fire_g(k + 2)

        @pl.when(k < NCHT - 5)
        def _fc():
            fire_c(k + 5)

        wait_g(p)

        def scale(g, _):
            wv = wbuf[q, pl.ds(g * 16, 16)]
            for j in range(16):
                e = g * 16 + j
                w = jnp.broadcast_to(wv[j], (16,))
                for dd in range(D // 16):
                    sl = pl.ds(dd * 16, 16)
                    rows[p, e, sl] = rows[p, e, sl] * w
            return 0

        lax.fori_loop(0, CH // 16, scale, 0)
        pltpu.async_copy(rows.at[p], acc.at[ebuf.at[q, 1]], ssem.at[p],
                         add=True)
        return 0

    wait_c(0)
    fire_g(0)
    wait_c(1)
    fire_g(1)
    lax.fori_loop(0, NCHT, body, 0)
    wait_s((NCHT - 2) % 4)
    wait_s((NCHT - 1) % 4)
    plsc.subcore_barrier()

    pltpu.sync_copy(acc.at[pl.ds(s * 640, 640)],
                    out_hbm.at[c].at[pl.ds(s * 640, 640)])


def _y_body(comp_ref, x_ref, basis_ref, y_ref):
    r = pl.program_id(1)
    rows = lax.broadcasted_iota(jnp.int32, (R, 1), 0)
    comp_r = jnp.sum(jnp.where(rows == r, comp_ref[...], 0.0), axis=0,
                     keepdims=True)
    w_r = jnp.dot(comp_r, basis_ref[...].reshape(NB, D * D),
                  preferred_element_type=jnp.float32).reshape(D, D)
    y_ref[0] = jnp.dot(x_ref[...], w_r, preferred_element_type=jnp.float32)


def _y_kernel(x, basis, comp):
    return pl.pallas_call(
        _y_body,
        grid=(N // BN, R),
        in_specs=[
            pl.BlockSpec((R, NB), lambda n, r: (0, 0)),
            pl.BlockSpec((BN, D), lambda n, r: (n, 0)),
            pl.BlockSpec((NB, D, D), lambda n, r: (0, 0, 0)),
        ],
        out_specs=pl.BlockSpec((1, BN, D), lambda n, r: (r, n, 0)),
        out_shape=jax.ShapeDtypeStruct((R, N, D), jnp.float32),
    )(comp, x, basis)


def _combine_body(x_ref, agg_ref, root_ref, bias_ref, lnw_ref, lnb_ref, out_ref):
    agg = agg_ref[0] + agg_ref[1]
    t = agg + jnp.dot(x_ref[...], root_ref[...],
                      preferred_element_type=jnp.float32) + bias_ref[...]
    mu = jnp.mean(t, axis=-1, keepdims=True)
    var = jnp.mean((t - mu) ** 2, axis=-1, keepdims=True)
    t = (t - mu) * lax.rsqrt(var + 1e-5) * lnw_ref[...] + lnb_ref[...]
    out_ref[...] = x_ref[...] + jnp.maximum(t, 0.0)


def _combine_kernel(x, agg2, root, bias, lnw, lnb):
    return pl.pallas_call(
        _combine_body,
        grid=(N // BN,),
        in_specs=[
            pl.BlockSpec((BN, D), lambda n: (n, 0)),
            pl.BlockSpec((2, BN, D), lambda n: (0, n, 0)),
            pl.BlockSpec((D, D), lambda n: (0, 0)),
            pl.BlockSpec((1, D), lambda n: (0, 0)),
            pl.BlockSpec((1, D), lambda n: (0, 0)),
            pl.BlockSpec((1, D), lambda n: (0, 0)),
        ],
        out_specs=pl.BlockSpec((BN, D), lambda n: (n, 0)),
        out_shape=jax.ShapeDtypeStruct((N, D), jnp.float32),
    )(x, agg2, root, bias.reshape(1, D), lnw.reshape(1, D), lnb.reshape(1, D))


def _mean_body(x_ref, out_ref):
    @pl.when(pl.program_id(0) == 0)
    def _init():
        out_ref[...] = jnp.zeros_like(out_ref)
    out_ref[...] += jnp.sum(x_ref[...], axis=0, keepdims=True) * (1.0 / N)


def _mean_kernel(x):
    return pl.pallas_call(
        _mean_body,
        grid=(N // BN,),
        in_specs=[pl.BlockSpec((BN, D), lambda n: (n, 0))],
        out_specs=pl.BlockSpec((1, D), lambda n: (0, 0)),
        out_shape=jax.ShapeDtypeStruct((1, D), jnp.float32),
    )(x)


def kernel(node_ids, edge_index, edge_type, emb,
           basis0, comp0, root0, bias0, lnw0, lnb0,
           basis1, comp1, root1, bias1, lnw1, lnb1,
           basis2, comp2, root2, bias2, lnw2, lnb2):
    src = edge_index[0]
    dst = edge_index[1]
    etype = edge_type

    pad = EPAD - E
    seg = jnp.concatenate([dst * R + etype,
                           jnp.full((pad,), N * R, jnp.int32)])
    gidx = jnp.concatenate([etype * N + src, jnp.zeros((pad,), jnp.int32)])
    dstp = jnp.concatenate([dst, jnp.full((pad,), N, jnp.int32)])
    ids = jnp.concatenate([node_ids,
                           jnp.zeros((NPAD - N,), node_ids.dtype)])

    w_edge = _sc_weights(seg)
    x = _sc_x0(emb, ids)[:N]

    edata = jnp.stack([gidx.reshape(-1, CH), dstp.reshape(-1, CH)], axis=1)

    layers = [(basis0, comp0, root0, bias0, lnw0, lnb0),
              (basis1, comp1, root1, bias1, lnw1, lnb1),
              (basis2, comp2, root2, bias2, lnw2, lnb2)]
    for (ba, co, ro, bi, lw, lb) in layers:
        y = _y_kernel(x, ba, co).reshape(R * N, D)
        agg2 = _sc_edge(y, edata, w_edge)
        x = _combine_kernel(x, agg2, ro, bi, lw, lb)

    return (x, _mean_kernel(x))
